# baseline (speedup 1.0000x reference)
import functools

import jax
import jax.numpy as jnp
from jax import lax
from jax.experimental import pallas as pl
from jax.experimental.pallas import tpu as pltpu
from jax.experimental.pallas import tpu_sc as plsc

NUM_NODES = 100000
D_FEAT = 128
EMB = 64
DEG = 16
B = 16384

NC = 2
NS = 16
NW = NC * NS

_PW = EMB // 2
_N4 = NUM_NODES // 4
_PROJ_R4 = 1000
_NBLK = _N4 // _PROJ_R4


def _pack_bf16(p):
    u = jax.lax.bitcast_convert_type(p, jnp.uint32)
    rnd = (u + 0x7FFF + ((u >> 16) & 1)) >> 16
    word = rnd[:, :_PW] | (rnd[:, _PW:] << 16)
    return jax.lax.bitcast_convert_type(word, jnp.int32)


def _proj_body(fw0_ref, fw1_ref, fw2_ref, fw3_ref, w_ref, b_ref, out_ref):
    w = w_ref[...]
    b = b_ref[...]
    packs = []
    for fw_ref in (fw0_ref, fw1_ref, fw2_ref, fw3_ref):
        p = jnp.dot(fw_ref[...], w, preferred_element_type=jnp.float32) + b
        packs.append(_pack_bf16(p))
    out_ref[...] = jnp.concatenate(packs, axis=1)


_NR8 = 12800
_NRB = 400
_NR_LAST = NUM_NODES // _NRB - 1


def _nref_body(n0, n1, n2, n3, n4, n5, n6, n7, out_ref):
    out_ref[...] = jnp.concatenate(
        [r[...] for r in (n0, n1, n2, n3, n4, n5, n6, n7)], axis=1
    )


def _reformat_neighs(neighs):
    nsteps = _NR8 // _NRB
    specs = [
        pl.BlockSpec(
            (_NRB, DEG),
            (lambda j: (
                lambda i, _j=j: (jnp.minimum(i + nsteps * _j, _NR_LAST), 0)
            ))(j),
        )
        for j in range(8)
    ]
    return pl.pallas_call(
        _nref_body,
        grid=(nsteps,),
        in_specs=specs,
        out_specs=pl.BlockSpec((_NRB, 8 * DEG), lambda i: (i, 0)),
        out_shape=jax.ShapeDtypeStruct((_NR8, 8 * DEG), jnp.int32),
    )(*([neighs] * 8))


def _project_table(feature_weight, W_l0, b_row):
    fw_specs = [
        pl.BlockSpec((_PROJ_R4, D_FEAT),
                     (lambda j: (lambda i, _j=j: (i + _NBLK * _j, 0)))(j))
        for j in range(4)
    ]
    return pl.pallas_call(
        _proj_body,
        grid=(_NBLK,),
        in_specs=fw_specs + [
            pl.BlockSpec((D_FEAT, EMB), lambda i: (0, 0)),
            pl.BlockSpec((1, EMB), lambda i: (0, 0)),
        ],
        out_specs=pl.BlockSpec((_PROJ_R4, 4 * _PW), lambda i: (i, 0)),
        out_shape=jax.ShapeDtypeStruct((_N4, 4 * _PW), jnp.int32),
    )(feature_weight, feature_weight, feature_weight, feature_weight,
      W_l0, b_row)


_BPW = B // NW
_CH = 128
_FPW = (B * DEG) // NW
_NCHUNKS = _FPW // _CH


@functools.cache
def _sc_kernels():
    mesh = plsc.VectorSubcoreMesh(core_axis_name="c", subcore_axis_name="s")
    params = pltpu.CompilerParams(
        use_tc_tiling_on_sc=False, needs_layout_passes=False
    )

    @functools.partial(
        pl.kernel,
        out_type=(
            jax.ShapeDtypeStruct((B, _PW), jnp.int32),
            jax.ShapeDtypeStruct((DEG, B, _PW), jnp.int32),
        ),
        mesh=mesh,
        compiler_params=params,
        scratch_types=[
            pltpu.VMEM((_BPW,), jnp.int32),
            pltpu.VMEM((_BPW,), jnp.int32),
            pltpu.VMEM((_BPW,), jnp.int32),
            pltpu.VMEM((_BPW, DEG), jnp.int32),
            pltpu.VMEM((_FPW,), jnp.int32),
            pltpu.VMEM((_CH, _PW), jnp.int32),
            pltpu.VMEM((_CH, _PW), jnp.int32),
            pltpu.SemaphoreType.DMA,
            pltpu.SemaphoreType.DMA,
            pltpu.SemaphoreType.DMA,
        ],
    )
    def fused_gather(nodes_hbm, neighs_hbm, proj_hbm, nf_hbm, gf_hbm,
                     idx_v, idx_q, idx_q8, nidx_v, nidx_t, rows0, rows1,
                     semi, sem0, sem1):
        wid = lax.axis_index("s") * NC + lax.axis_index("c")
        base = wid * _BPW

        def to_row(v):
            j = ((v >= _N4).astype(jnp.int32)
                 + (v >= 2 * _N4).astype(jnp.int32)
                 + (v >= 3 * _N4).astype(jnp.int32))
            return (v - j * _N4) * 4 + j

        def to_nrow(v):
            j = (v >= _NR8).astype(jnp.int32)
            for t in range(2, 8):
                j = j + (v >= t * _NR8).astype(jnp.int32)
            return (v - j * _NR8) * 8 + j

        pltpu.sync_copy(nodes_hbm.at[pl.ds(base, _BPW)], idx_v)

        def qbody(i, _):
            v = idx_v[pl.ds(16 * i, 16)]
            idx_q[pl.ds(16 * i, 16)] = to_row(v)
            idx_q8[pl.ds(16 * i, 16)] = to_nrow(v)
            return 0

        lax.fori_loop(0, _BPW // 16, qbody, 0)
        for c in range(_BPW // _CH):
            pltpu.async_copy(
                neighs_hbm.at[idx_q8.at[pl.ds(c * _CH, _CH)]],
                nidx_v.at[pl.ds(c * _CH, _CH)],
                semi,
            )
        for c in range(_BPW // _CH):
            pltpu.make_async_copy(
                neighs_hbm.at[idx_q8.at[pl.ds(c * _CH, _CH)]],
                nidx_v.at[pl.ds(c * _CH, _CH)],
                semi,
            ).wait()
        lanes = lax.iota(jnp.int32, 16)

        def tbody(j, _):
            rows = 16 * j + lanes
            for k in range(DEG):
                v = plsc.load_gather(
                    nidx_v, [rows, jnp.full((16,), k, jnp.int32)]
                )
                nidx_t[pl.ds(k * _BPW + 16 * j, 16)] = to_row(v)
            return 0

        lax.fori_loop(0, _BPW // 16, tbody, 0)

        def issue(idx_ref, ioff, buf, sem):
            pltpu.async_copy(
                proj_hbm.at[idx_ref.at[pl.ds(ioff, _CH)]], buf, sem
            )

        def drain(idx_ref, ioff, buf, sem):
            pltpu.make_async_copy(
                proj_hbm.at[idx_ref.at[pl.ds(ioff, _CH)]], buf, sem
            ).wait()

        def gf_dst(c):
            kd = c // (_BPW // _CH)
            boff = base + (c % (_BPW // _CH)) * _CH
            return gf_hbm.at[kd, pl.ds(boff, _CH)]

        bufs = (rows0, rows1)
        sems = (sem0, sem1)
        ncn = _BPW // _CH

        issue(idx_q, 0, bufs[0], sems[0])
        issue(idx_q, _CH, bufs[1], sems[1])
        for c in range(2, ncn):
            p = c % 2
            drain(idx_q, (c - 2) * _CH, bufs[p], sems[p])
            pltpu.sync_copy(bufs[p], nf_hbm.at[pl.ds(base + (c - 2) * _CH, _CH)])
            issue(idx_q, c * _CH, bufs[p], sems[p])
        for c in range(ncn - 2, ncn):
            p = c % 2
            drain(idx_q, c * _CH, bufs[p], sems[p])
            pltpu.sync_copy(bufs[p], nf_hbm.at[pl.ds(base + c * _CH, _CH)])
            issue(nidx_t, (c - ncn + 2) * _CH, bufs[p], sems[p])

        def body(i, _):
            c0 = 2 * i
            drain(nidx_t, c0 * _CH, rows0, sem0)
            pltpu.sync_copy(rows0, gf_dst(c0))

            @pl.when(c0 + 2 < _NCHUNKS)
            def _():
                issue(nidx_t, (c0 + 2) * _CH, rows0, sem0)

            drain(nidx_t, (c0 + 1) * _CH, rows1, sem1)
            pltpu.sync_copy(rows1, gf_dst(c0 + 1))

            @pl.when(c0 + 3 < _NCHUNKS)
            def _():
                issue(nidx_t, (c0 + 3) * _CH, rows1, sem1)

            return 0

        lax.fori_loop(0, _NCHUNKS // 2, body, 0)

    return fused_gather


_B4 = B // 4
_TBR = 256


def _unpack4(w):
    lo = jax.lax.bitcast_convert_type(w << 16, jnp.float32)
    hi = jax.lax.bitcast_convert_type(w & jnp.int32(-65536), jnp.float32)
    return jnp.concatenate([lo, hi], axis=1), lo, hi


def _att_body(nf_ref, gf_ref, w1n4_ref, w1s4_ref, b1r_ref, w24_ref,
              sel_ref, rep_ref, b2_ref, lo_ref, hi_ref, e_mem):
    nlane, _, _ = _unpack4(nf_ref[...])
    s4 = (
        jnp.dot(nlane.astype(jnp.bfloat16), w1s4_ref[...],
                preferred_element_type=jnp.float32)
        + b1r_ref[...]
    )
    w1n4 = w1n4_ref[...]
    w24 = w24_ref[...]
    b2 = b2_ref[0, 0]
    for k in range(DEG):
        glane, _, _ = _unpack4(gf_ref[k])
        h = jnp.maximum(
            jnp.dot(glane.astype(jnp.bfloat16), w1n4,
                    preferred_element_type=jnp.float32) + s4,
            0.0,
        )
        l4 = jnp.dot(h, w24, preferred_element_type=jnp.float32) + b2
        e_mem[:, 4 * k:4 * (k + 1)] = jnp.exp(l4)
    e_all = e_mem[...]
    denom = jnp.dot(e_all, sel_ref[...],
                    preferred_element_type=jnp.float32)
    inv = 1.0 / denom
    rep = rep_ref[...]
    acc_lo = None
    acc_hi = None
    for k in range(DEG):
        aw = e_all[:, 4 * k:4 * (k + 1)] * inv
        awb = jnp.dot(aw, rep, preferred_element_type=jnp.float32)
        g = gf_ref[k]
        lo = jax.lax.bitcast_convert_type(g << 16, jnp.float32)
        hi = jax.lax.bitcast_convert_type(g & jnp.int32(-65536), jnp.float32)
        if acc_lo is None:
            acc_lo = awb * lo
            acc_hi = awb * hi
        else:
            acc_lo = acc_lo + awb * lo
            acc_hi = acc_hi + awb * hi
    lo_ref[...] = acc_lo
    hi_ref[...] = acc_hi


def _attention(nf4, gf4, w1n4, w1s4, b1r, w24, sel, rep, b2_sq):
    return pl.pallas_call(
        _att_body,
        grid=(_B4 // _TBR,),
        in_specs=[
            pl.BlockSpec((_TBR, 128), lambda i: (i, 0)),
            pl.BlockSpec((DEG, _TBR, 128), lambda i: (0, i, 0)),
            pl.BlockSpec((256, 256), lambda i: (0, 0)),
            pl.BlockSpec((256, 256), lambda i: (0, 0)),
            pl.BlockSpec((1, 256), lambda i: (0, 0)),
            pl.BlockSpec((256, 4), lambda i: (0, 0)),
            pl.BlockSpec((EMB, 4), lambda i: (0, 0)),
            pl.BlockSpec((4, 128), lambda i: (0, 0)),
            pl.BlockSpec((1, 1), lambda i: (0, 0)),
        ],
        out_specs=[
            pl.BlockSpec((_TBR, 128), lambda i: (i, 0)),
            pl.BlockSpec((_TBR, 128), lambda i: (i, 0)),
        ],
        out_shape=[
            jax.ShapeDtypeStruct((_B4, 128), jnp.float32),
            jax.ShapeDtypeStruct((_B4, 128), jnp.float32),
        ],
        scratch_shapes=[pltpu.VMEM((_TBR, EMB), jnp.float32)],
    )(nf4, gf4, w1n4, w1s4, b1r, w24, sel, rep, b2_sq)


def kernel(nodes, neighs, feature_weight, W_l0, b_l0, att_W1, att_b1,
           att_W2, att_b2):
    fused_gather = _sc_kernels()
    proj4 = _project_table(feature_weight, W_l0, b_l0.reshape(1, EMB))
    proj = proj4.reshape(NUM_NODES, _PW)
    nr8 = _reformat_neighs(neighs)
    neighs_r = nr8.reshape(8 * _NR8, DEG)
    nf, gf = fused_gather(nodes, neighs_r, proj)
    nf4 = nf.reshape(_B4, 128)
    gf4 = gf.reshape(DEG, _B4, 128)
    eye4 = jnp.eye(4, dtype=jnp.float32)
    w1n = att_W1[:EMB]
    w1s = att_W1[EMB:]
    w1n4 = jnp.concatenate(
        [jnp.kron(eye4, w1n[:_PW]), jnp.kron(eye4, w1n[_PW:])], axis=0
    ).astype(jnp.bfloat16)
    w1s4 = jnp.concatenate(
        [jnp.kron(eye4, w1s[:_PW]), jnp.kron(eye4, w1s[_PW:])], axis=0
    ).astype(jnp.bfloat16)
    b1r = jnp.tile(att_b1.reshape(1, EMB), (1, 4))
    w24 = jnp.kron(eye4, att_W2)
    sel = jnp.kron(jnp.ones((DEG, 1), jnp.float32), eye4)
    rep = jnp.kron(eye4, jnp.ones((1, 32), jnp.float32))
    lo, hi = _attention(nf4, gf4, w1n4, w1s4, b1r, w24, sel, rep,
                        att_b2.reshape(1, 1))
    return jnp.concatenate(
        [lo.reshape(B, _PW), hi.reshape(B, _PW)], axis=1
    )


# speedup vs baseline: 1.1936x; 1.1936x over previous
import functools

import jax
import jax.numpy as jnp
from jax import lax
from jax.experimental import pallas as pl
from jax.experimental.pallas import tpu as pltpu
from jax.experimental.pallas import tpu_sc as plsc

NUM_NODES = 100000
D_FEAT = 128
EMB = 64
DEG = 16
B = 16384

NC = 2
NS = 16
NW = NC * NS

_PW = EMB // 2
_N4 = NUM_NODES // 4
_PROJ_R4 = 5000
_NBLK = _N4 // _PROJ_R4


def _pack_bf16(p):
    u = jax.lax.bitcast_convert_type(p, jnp.uint32)
    rnd = (u + 0x7FFF + ((u >> 16) & 1)) >> 16
    word = rnd[:, :_PW] | (rnd[:, _PW:] << 16)
    return jax.lax.bitcast_convert_type(word, jnp.int32)


def _proj_body(fw0_ref, fw1_ref, fw2_ref, fw3_ref, w_ref, b_ref, out_ref):
    w = w_ref[...]
    b = b_ref[...]
    packs = []
    for fw_ref in (fw0_ref, fw1_ref, fw2_ref, fw3_ref):
        p = jnp.dot(fw_ref[...], w, preferred_element_type=jnp.float32) + b
        packs.append(_pack_bf16(p))
    out_ref[...] = jnp.concatenate(packs, axis=1)


_NR8 = 12800
_NRB = 1600


def _nref_body(n0, n1, n2, n3, n4, n5, n6, n7, out_ref):
    out_ref[...] = jnp.concatenate(
        [r[...] for r in (n0, n1, n2, n3, n4, n5, n6, n7)], axis=1
    )


def _reformat_neighs(neighs):
    specs = [
        pl.BlockSpec((_NRB, DEG),
                     (lambda j: (lambda i, _j=j: (i + 8 * _j, 0)))(j))
        for j in range(8)
    ]
    return pl.pallas_call(
        _nref_body,
        grid=(_NR8 // _NRB,),
        in_specs=specs,
        out_specs=pl.BlockSpec((_NRB, 8 * DEG), lambda i: (i, 0)),
        out_shape=jax.ShapeDtypeStruct((_NR8, 8 * DEG), jnp.int32),
    )(*([neighs] * 8))


def _project_table(feature_weight, W_l0, b_row):
    fw_specs = [
        pl.BlockSpec((_PROJ_R4, D_FEAT),
                     (lambda j: (lambda i, _j=j: (i + _NBLK * _j, 0)))(j))
        for j in range(4)
    ]
    return pl.pallas_call(
        _proj_body,
        grid=(_NBLK,),
        in_specs=fw_specs + [
            pl.BlockSpec((D_FEAT, EMB), lambda i: (0, 0)),
            pl.BlockSpec((1, EMB), lambda i: (0, 0)),
        ],
        out_specs=pl.BlockSpec((_PROJ_R4, 4 * _PW), lambda i: (i, 0)),
        out_shape=jax.ShapeDtypeStruct((_N4, 4 * _PW), jnp.int32),
    )(feature_weight, feature_weight, feature_weight, feature_weight,
      W_l0, b_row)


_BPW = B // NW
_CH = 128
_FPW = (B * DEG) // NW
_NCHUNKS = _FPW // _CH


@functools.cache
def _sc_kernels():
    mesh = plsc.VectorSubcoreMesh(core_axis_name="c", subcore_axis_name="s")
    params = pltpu.CompilerParams(
        use_tc_tiling_on_sc=False, needs_layout_passes=False
    )

    @functools.partial(
        pl.kernel,
        out_type=(
            jax.ShapeDtypeStruct((B, _PW), jnp.int32),
            jax.ShapeDtypeStruct((DEG, B, _PW), jnp.int32),
        ),
        mesh=mesh,
        compiler_params=params,
        scratch_types=[
            pltpu.VMEM((_BPW,), jnp.int32),
            pltpu.VMEM((_BPW,), jnp.int32),
            pltpu.VMEM((_BPW,), jnp.int32),
            pltpu.VMEM((_BPW, DEG), jnp.int32),
            pltpu.VMEM((_FPW,), jnp.int32),
            pltpu.VMEM((_CH, _PW), jnp.int32),
            pltpu.VMEM((_CH, _PW), jnp.int32),
            pltpu.SemaphoreType.DMA,
            pltpu.SemaphoreType.DMA,
            pltpu.SemaphoreType.DMA,
        ],
    )
    def fused_gather(nodes_hbm, neighs_hbm, proj_hbm, nf_hbm, gf_hbm,
                     idx_v, idx_q, idx_q8, nidx_v, nidx_t, rows0, rows1,
                     semi, sem0, sem1):
        wid = lax.axis_index("s") * NC + lax.axis_index("c")
        base = wid * _BPW

        def to_row(v):
            j = ((v >= _N4).astype(jnp.int32)
                 + (v >= 2 * _N4).astype(jnp.int32)
                 + (v >= 3 * _N4).astype(jnp.int32))
            return (v - j * _N4) * 4 + j

        def to_nrow(v):
            j = (v >= _NR8).astype(jnp.int32)
            for t in range(2, 8):
                j = j + (v >= t * _NR8).astype(jnp.int32)
            return (v - j * _NR8) * 8 + j

        pltpu.sync_copy(nodes_hbm.at[pl.ds(base, _BPW)], idx_v)

        def qbody(i, _):
            v = idx_v[pl.ds(16 * i, 16)]
            idx_q[pl.ds(16 * i, 16)] = to_row(v)
            return 0

        lax.fori_loop(0, _BPW // 16, qbody, 0)
        for c in range(_BPW // _CH):
            pltpu.async_copy(
                neighs_hbm.at[idx_v.at[pl.ds(c * _CH, _CH)]],
                nidx_v.at[pl.ds(c * _CH, _CH)],
                semi,
            )
        for c in range(_BPW // _CH):
            pltpu.make_async_copy(
                neighs_hbm.at[idx_v.at[pl.ds(c * _CH, _CH)]],
                nidx_v.at[pl.ds(c * _CH, _CH)],
                semi,
            ).wait()
        lanes = lax.iota(jnp.int32, 16)

        def tbody(j, _):
            rows = 16 * j + lanes
            for k in range(DEG):
                v = plsc.load_gather(
                    nidx_v, [rows, jnp.full((16,), k, jnp.int32)]
                )
                nidx_t[pl.ds(k * _BPW + 16 * j, 16)] = to_row(v)
            return 0

        lax.fori_loop(0, _BPW // 16, tbody, 0)

        def issue(idx_ref, ioff, buf, sem):
            pltpu.async_copy(
                proj_hbm.at[idx_ref.at[pl.ds(ioff, _CH)]], buf, sem
            )

        def drain(idx_ref, ioff, buf, sem):
            pltpu.make_async_copy(
                proj_hbm.at[idx_ref.at[pl.ds(ioff, _CH)]], buf, sem
            ).wait()

        def gf_dst(c):
            kd = c // (_BPW // _CH)
            boff = base + (c % (_BPW // _CH)) * _CH
            return gf_hbm.at[kd, pl.ds(boff, _CH)]

        bufs = (rows0, rows1)
        sems = (sem0, sem1)
        ncn = _BPW // _CH

        issue(idx_q, 0, bufs[0], sems[0])
        issue(idx_q, _CH, bufs[1], sems[1])
        for c in range(2, ncn):
            p = c % 2
            drain(idx_q, (c - 2) * _CH, bufs[p], sems[p])
            pltpu.sync_copy(bufs[p], nf_hbm.at[pl.ds(base + (c - 2) * _CH, _CH)])
            issue(idx_q, c * _CH, bufs[p], sems[p])
        for c in range(ncn - 2, ncn):
            p = c % 2
            drain(idx_q, c * _CH, bufs[p], sems[p])
            pltpu.sync_copy(bufs[p], nf_hbm.at[pl.ds(base + c * _CH, _CH)])
            issue(nidx_t, (c - ncn + 2) * _CH, bufs[p], sems[p])

        def body(i, _):
            c0 = 2 * i
            drain(nidx_t, c0 * _CH, rows0, sem0)
            pltpu.sync_copy(rows0, gf_dst(c0))

            @pl.when(c0 + 2 < _NCHUNKS)
            def _():
                issue(nidx_t, (c0 + 2) * _CH, rows0, sem0)

            drain(nidx_t, (c0 + 1) * _CH, rows1, sem1)
            pltpu.sync_copy(rows1, gf_dst(c0 + 1))

            @pl.when(c0 + 3 < _NCHUNKS)
            def _():
                issue(nidx_t, (c0 + 3) * _CH, rows1, sem1)

            return 0

        lax.fori_loop(0, _NCHUNKS // 2, body, 0)

    return fused_gather


_B4 = B // 4
_TBR = 512


def _unpack4(w):
    lo = jax.lax.bitcast_convert_type(w << 16, jnp.float32)
    hi = jax.lax.bitcast_convert_type(w & jnp.int32(-65536), jnp.float32)
    return jnp.concatenate([lo, hi], axis=1), lo, hi


def _att_body(nf_ref, gf_ref, w1n4_ref, w1s4_ref, b1r_ref, w24_ref,
              sel_ref, rep_ref, b2_ref, lo_ref, hi_ref, e_mem):
    nlane, _, _ = _unpack4(nf_ref[...])
    s4 = (
        jnp.dot(nlane.astype(jnp.bfloat16), w1s4_ref[...],
                preferred_element_type=jnp.float32)
        + b1r_ref[...]
    )
    w1n4 = w1n4_ref[...]
    w24 = w24_ref[...]
    b2 = b2_ref[0, 0]
    for k in range(DEG):
        glane, _, _ = _unpack4(gf_ref[k])
        h = jnp.maximum(
            jnp.dot(glane.astype(jnp.bfloat16), w1n4,
                    preferred_element_type=jnp.float32) + s4,
            0.0,
        )
        l4 = jnp.dot(h, w24, preferred_element_type=jnp.float32) + b2
        e_mem[:, 4 * k:4 * (k + 1)] = jnp.exp(l4)
    e_all = e_mem[...]
    denom = jnp.dot(e_all, sel_ref[...],
                    preferred_element_type=jnp.float32)
    inv = 1.0 / denom
    rep = rep_ref[...]
    acc_lo = None
    acc_hi = None
    for k in range(DEG):
        aw = e_all[:, 4 * k:4 * (k + 1)] * inv
        awb = jnp.dot(aw, rep, preferred_element_type=jnp.float32)
        g = gf_ref[k]
        lo = jax.lax.bitcast_convert_type(g << 16, jnp.float32)
        hi = jax.lax.bitcast_convert_type(g & jnp.int32(-65536), jnp.float32)
        if acc_lo is None:
            acc_lo = awb * lo
            acc_hi = awb * hi
        else:
            acc_lo = acc_lo + awb * lo
            acc_hi = acc_hi + awb * hi
    lo_ref[...] = acc_lo
    hi_ref[...] = acc_hi


def _attention(nf4, gf4, w1n4, w1s4, b1r, w24, sel, rep, b2_sq):
    return pl.pallas_call(
        _att_body,
        grid=(_B4 // _TBR,),
        in_specs=[
            pl.BlockSpec((_TBR, 128), lambda i: (i, 0)),
            pl.BlockSpec((DEG, _TBR, 128), lambda i: (0, i, 0)),
            pl.BlockSpec((256, 256), lambda i: (0, 0)),
            pl.BlockSpec((256, 256), lambda i: (0, 0)),
            pl.BlockSpec((1, 256), lambda i: (0, 0)),
            pl.BlockSpec((256, 4), lambda i: (0, 0)),
            pl.BlockSpec((EMB, 4), lambda i: (0, 0)),
            pl.BlockSpec((4, 128), lambda i: (0, 0)),
            pl.BlockSpec((1, 1), lambda i: (0, 0)),
        ],
        out_specs=[
            pl.BlockSpec((_TBR, 128), lambda i: (i, 0)),
            pl.BlockSpec((_TBR, 128), lambda i: (i, 0)),
        ],
        out_shape=[
            jax.ShapeDtypeStruct((_B4, 128), jnp.float32),
            jax.ShapeDtypeStruct((_B4, 128), jnp.float32),
        ],
        scratch_shapes=[pltpu.VMEM((_TBR, EMB), jnp.float32)],
    )(nf4, gf4, w1n4, w1s4, b1r, w24, sel, rep, b2_sq)


def kernel(nodes, neighs, feature_weight, W_l0, b_l0, att_W1, att_b1,
           att_W2, att_b2):
    fused_gather = _sc_kernels()
    proj4 = _project_table(feature_weight, W_l0, b_l0.reshape(1, EMB))
    proj = proj4.reshape(NUM_NODES, _PW)
    nf, gf = fused_gather(nodes, neighs, proj)
    nf4 = nf.reshape(_B4, 128)
    gf4 = gf.reshape(DEG, _B4, 128)
    eye4 = jnp.eye(4, dtype=jnp.float32)
    w1n = att_W1[:EMB]
    w1s = att_W1[EMB:]
    w1n4 = jnp.concatenate(
        [jnp.kron(eye4, w1n[:_PW]), jnp.kron(eye4, w1n[_PW:])], axis=0
    ).astype(jnp.bfloat16)
    w1s4 = jnp.concatenate(
        [jnp.kron(eye4, w1s[:_PW]), jnp.kron(eye4, w1s[_PW:])], axis=0
    ).astype(jnp.bfloat16)
    b1r = jnp.tile(att_b1.reshape(1, EMB), (1, 4))
    w24 = jnp.kron(eye4, att_W2)
    sel = jnp.kron(jnp.ones((DEG, 1), jnp.float32), eye4)
    rep = jnp.kron(eye4, jnp.ones((1, 32), jnp.float32))
    lo, hi = _attention(nf4, gf4, w1n4, w1s4, b1r, w24, sel, rep,
                        att_b2.reshape(1, 1))
    return jnp.concatenate(
        [lo.reshape(B, _PW), hi.reshape(B, _PW)], axis=1
    )


# speedup vs baseline: 1.1990x; 1.0045x over previous
import functools

import jax
import jax.numpy as jnp
from jax import lax
from jax.experimental import pallas as pl
from jax.experimental.pallas import tpu as pltpu
from jax.experimental.pallas import tpu_sc as plsc

NUM_NODES = 100000
D_FEAT = 128
EMB = 64
DEG = 16
B = 16384

NC = 2
NS = 16
NW = NC * NS

_PW = EMB // 2
_N4 = NUM_NODES // 4
_PROJ_R4 = 5000
_NBLK = _N4 // _PROJ_R4


def _pack_bf16(p):
    u = jax.lax.bitcast_convert_type(p, jnp.uint32)
    rnd = (u + 0x7FFF + ((u >> 16) & 1)) >> 16
    word = rnd[:, :_PW] | (rnd[:, _PW:] << 16)
    return jax.lax.bitcast_convert_type(word, jnp.int32)


def _proj_body(fw0_ref, fw1_ref, fw2_ref, fw3_ref, w_ref, b_ref, out_ref):
    w = w_ref[...]
    b = b_ref[...]
    packs = []
    for fw_ref in (fw0_ref, fw1_ref, fw2_ref, fw3_ref):
        p = jnp.dot(fw_ref[...], w, preferred_element_type=jnp.float32) + b
        packs.append(_pack_bf16(p))
    out_ref[...] = jnp.concatenate(packs, axis=1)


_NR8 = 12800
_NRB = 1600


def _nref_body(n0, n1, n2, n3, n4, n5, n6, n7, out_ref):
    out_ref[...] = jnp.concatenate(
        [r[...] for r in (n0, n1, n2, n3, n4, n5, n6, n7)], axis=1
    )


def _reformat_neighs(neighs):
    specs = [
        pl.BlockSpec((_NRB, DEG),
                     (lambda j: (lambda i, _j=j: (i + 8 * _j, 0)))(j))
        for j in range(8)
    ]
    return pl.pallas_call(
        _nref_body,
        grid=(_NR8 // _NRB,),
        in_specs=specs,
        out_specs=pl.BlockSpec((_NRB, 8 * DEG), lambda i: (i, 0)),
        out_shape=jax.ShapeDtypeStruct((_NR8, 8 * DEG), jnp.int32),
    )(*([neighs] * 8))


def _project_table(feature_weight, W_l0, b_row):
    fw_specs = [
        pl.BlockSpec((_PROJ_R4, D_FEAT),
                     (lambda j: (lambda i, _j=j: (i + _NBLK * _j, 0)))(j))
        for j in range(4)
    ]
    return pl.pallas_call(
        _proj_body,
        grid=(_NBLK,),
        in_specs=fw_specs + [
            pl.BlockSpec((D_FEAT, EMB), lambda i: (0, 0)),
            pl.BlockSpec((1, EMB), lambda i: (0, 0)),
        ],
        out_specs=pl.BlockSpec((_PROJ_R4, 4 * _PW), lambda i: (i, 0)),
        out_shape=jax.ShapeDtypeStruct((_N4, 4 * _PW), jnp.int32),
    )(feature_weight, feature_weight, feature_weight, feature_weight,
      W_l0, b_row)


_BPW = B // NW
_CH = 128
_FPW = (B * DEG) // NW
_NCHUNKS = _FPW // _CH


@functools.cache
def _sc_kernels():
    mesh = plsc.VectorSubcoreMesh(core_axis_name="c", subcore_axis_name="s")
    params = pltpu.CompilerParams(
        use_tc_tiling_on_sc=False, needs_layout_passes=False
    )

    @functools.partial(
        pl.kernel,
        out_type=(
            jax.ShapeDtypeStruct((B, _PW), jnp.int32),
            jax.ShapeDtypeStruct((DEG, B, _PW), jnp.int32),
        ),
        mesh=mesh,
        compiler_params=params,
        scratch_types=[
            pltpu.VMEM((_BPW,), jnp.int32),
            pltpu.VMEM((_BPW,), jnp.int32),
            pltpu.VMEM((_BPW,), jnp.int32),
            pltpu.VMEM((_BPW, DEG), jnp.int32),
            pltpu.VMEM((_FPW,), jnp.int32),
            pltpu.VMEM((_CH, _PW), jnp.int32),
            pltpu.VMEM((_CH, _PW), jnp.int32),
            pltpu.SemaphoreType.DMA,
            pltpu.SemaphoreType.DMA,
            pltpu.SemaphoreType.DMA,
        ],
    )
    def fused_gather(nodes_hbm, neighs_hbm, proj_hbm, nf_hbm, gf_hbm,
                     idx_v, idx_q, idx_q8, nidx_v, nidx_t, rows0, rows1,
                     semi, sem0, sem1):
        wid = lax.axis_index("s") * NC + lax.axis_index("c")
        base = wid * _BPW

        def to_row(v):
            j = ((v >= _N4).astype(jnp.int32)
                 + (v >= 2 * _N4).astype(jnp.int32)
                 + (v >= 3 * _N4).astype(jnp.int32))
            return (v - j * _N4) * 4 + j

        def to_nrow(v):
            j = (v >= _NR8).astype(jnp.int32)
            for t in range(2, 8):
                j = j + (v >= t * _NR8).astype(jnp.int32)
            return (v - j * _NR8) * 8 + j

        pltpu.sync_copy(nodes_hbm.at[pl.ds(base, _BPW)], idx_v)

        def qbody(i, _):
            v = idx_v[pl.ds(16 * i, 16)]
            idx_q[pl.ds(16 * i, 16)] = to_row(v)
            return 0

        lax.fori_loop(0, _BPW // 16, qbody, 0)
        for c in range(_BPW // _CH):
            pltpu.async_copy(
                neighs_hbm.at[idx_v.at[pl.ds(c * _CH, _CH)]],
                nidx_v.at[pl.ds(c * _CH, _CH)],
                semi,
            )
        for c in range(_BPW // _CH):
            pltpu.make_async_copy(
                neighs_hbm.at[idx_v.at[pl.ds(c * _CH, _CH)]],
                nidx_v.at[pl.ds(c * _CH, _CH)],
                semi,
            ).wait()
        lanes = lax.iota(jnp.int32, 16)

        def tbody(j, _):
            rows = 16 * j + lanes
            for k in range(DEG):
                v = plsc.load_gather(
                    nidx_v, [rows, jnp.full((16,), k, jnp.int32)]
                )
                nidx_t[pl.ds(k * _BPW + 16 * j, 16)] = to_row(v)
            return 0

        lax.fori_loop(0, _BPW // 16, tbody, 0)

        def issue(idx_ref, ioff, buf, sem):
            pltpu.async_copy(
                proj_hbm.at[idx_ref.at[pl.ds(ioff, _CH)]], buf, sem
            )

        def drain(idx_ref, ioff, buf, sem):
            pltpu.make_async_copy(
                proj_hbm.at[idx_ref.at[pl.ds(ioff, _CH)]], buf, sem
            ).wait()

        def gf_dst(c):
            kd = c // (_BPW // _CH)
            boff = base + (c % (_BPW // _CH)) * _CH
            return gf_hbm.at[kd, pl.ds(boff, _CH)]

        bufs = (rows0, rows1)
        sems = (sem0, sem1)
        ncn = _BPW // _CH

        issue(idx_q, 0, bufs[0], sems[0])
        issue(idx_q, _CH, bufs[1], sems[1])
        for c in range(2, ncn):
            p = c % 2
            drain(idx_q, (c - 2) * _CH, bufs[p], sems[p])
            pltpu.sync_copy(bufs[p], nf_hbm.at[pl.ds(base + (c - 2) * _CH, _CH)])
            issue(idx_q, c * _CH, bufs[p], sems[p])
        for c in range(ncn - 2, ncn):
            p = c % 2
            drain(idx_q, c * _CH, bufs[p], sems[p])
            pltpu.sync_copy(bufs[p], nf_hbm.at[pl.ds(base + c * _CH, _CH)])
            issue(nidx_t, (c - ncn + 2) * _CH, bufs[p], sems[p])

        def body(i, _):
            c0 = 2 * i
            drain(nidx_t, c0 * _CH, rows0, sem0)
            pltpu.sync_copy(rows0, gf_dst(c0))

            @pl.when(c0 + 2 < _NCHUNKS)
            def _():
                issue(nidx_t, (c0 + 2) * _CH, rows0, sem0)

            drain(nidx_t, (c0 + 1) * _CH, rows1, sem1)
            pltpu.sync_copy(rows1, gf_dst(c0 + 1))

            @pl.when(c0 + 3 < _NCHUNKS)
            def _():
                issue(nidx_t, (c0 + 3) * _CH, rows1, sem1)

            return 0

        lax.fori_loop(0, _NCHUNKS // 2, body, 0)

    return fused_gather


_B4 = B // 4
_TBR = 1024


def _unpack4(w):
    lo = jax.lax.bitcast_convert_type(w << 16, jnp.float32)
    hi = jax.lax.bitcast_convert_type(w & jnp.int32(-65536), jnp.float32)
    return jnp.concatenate([lo, hi], axis=1), lo, hi


def _att_body(nf_ref, gf_ref, w1n4_ref, w1s4_ref, b1r_ref, w24_ref,
              sel_ref, rep_ref, b2_ref, lo_ref, hi_ref, e_mem):
    nlane, _, _ = _unpack4(nf_ref[...])
    s4 = (
        jnp.dot(nlane.astype(jnp.bfloat16), w1s4_ref[...],
                preferred_element_type=jnp.float32)
        + b1r_ref[...]
    )
    w1n4 = w1n4_ref[...]
    w24 = w24_ref[...]
    b2 = b2_ref[0, 0]
    for k in range(DEG):
        glane, _, _ = _unpack4(gf_ref[k])
        h = jnp.maximum(
            jnp.dot(glane.astype(jnp.bfloat16), w1n4,
                    preferred_element_type=jnp.float32) + s4,
            0.0,
        )
        l4 = jnp.dot(h, w24, preferred_element_type=jnp.float32) + b2
        e_mem[:, 4 * k:4 * (k + 1)] = jnp.exp(l4)
    e_all = e_mem[...]
    denom = jnp.dot(e_all, sel_ref[...],
                    preferred_element_type=jnp.float32)
    inv = 1.0 / denom
    rep = rep_ref[...]
    acc_lo = None
    acc_hi = None
    for k in range(DEG):
        aw = e_all[:, 4 * k:4 * (k + 1)] * inv
        awb = jnp.dot(aw, rep, preferred_element_type=jnp.float32)
        g = gf_ref[k]
        lo = jax.lax.bitcast_convert_type(g << 16, jnp.float32)
        hi = jax.lax.bitcast_convert_type(g & jnp.int32(-65536), jnp.float32)
        if acc_lo is None:
            acc_lo = awb * lo
            acc_hi = awb * hi
        else:
            acc_lo = acc_lo + awb * lo
            acc_hi = acc_hi + awb * hi
    lo_ref[...] = acc_lo
    hi_ref[...] = acc_hi


def _attention(nf4, gf4, w1n4, w1s4, b1r, w24, sel, rep, b2_sq):
    return pl.pallas_call(
        _att_body,
        grid=(_B4 // _TBR,),
        in_specs=[
            pl.BlockSpec((_TBR, 128), lambda i: (i, 0)),
            pl.BlockSpec((DEG, _TBR, 128), lambda i: (0, i, 0)),
            pl.BlockSpec((256, 256), lambda i: (0, 0)),
            pl.BlockSpec((256, 256), lambda i: (0, 0)),
            pl.BlockSpec((1, 256), lambda i: (0, 0)),
            pl.BlockSpec((256, 4), lambda i: (0, 0)),
            pl.BlockSpec((EMB, 4), lambda i: (0, 0)),
            pl.BlockSpec((4, 128), lambda i: (0, 0)),
            pl.BlockSpec((1, 1), lambda i: (0, 0)),
        ],
        out_specs=[
            pl.BlockSpec((_TBR, 128), lambda i: (i, 0)),
            pl.BlockSpec((_TBR, 128), lambda i: (i, 0)),
        ],
        out_shape=[
            jax.ShapeDtypeStruct((_B4, 128), jnp.float32),
            jax.ShapeDtypeStruct((_B4, 128), jnp.float32),
        ],
        scratch_shapes=[pltpu.VMEM((_TBR, EMB), jnp.float32)],
    )(nf4, gf4, w1n4, w1s4, b1r, w24, sel, rep, b2_sq)


def kernel(nodes, neighs, feature_weight, W_l0, b_l0, att_W1, att_b1,
           att_W2, att_b2):
    fused_gather = _sc_kernels()
    proj4 = _project_table(feature_weight, W_l0, b_l0.reshape(1, EMB))
    proj = proj4.reshape(NUM_NODES, _PW)
    nf, gf = fused_gather(nodes, neighs, proj)
    nf4 = nf.reshape(_B4, 128)
    gf4 = gf.reshape(DEG, _B4, 128)
    eye4 = jnp.eye(4, dtype=jnp.float32)
    w1n = att_W1[:EMB]
    w1s = att_W1[EMB:]
    w1n4 = jnp.concatenate(
        [jnp.kron(eye4, w1n[:_PW]), jnp.kron(eye4, w1n[_PW:])], axis=0
    ).astype(jnp.bfloat16)
    w1s4 = jnp.concatenate(
        [jnp.kron(eye4, w1s[:_PW]), jnp.kron(eye4, w1s[_PW:])], axis=0
    ).astype(jnp.bfloat16)
    b1r = jnp.tile(att_b1.reshape(1, EMB), (1, 4))
    w24 = jnp.kron(eye4, att_W2)
    sel = jnp.kron(jnp.ones((DEG, 1), jnp.float32), eye4)
    rep = jnp.kron(eye4, jnp.ones((1, 32), jnp.float32))
    lo, hi = _attention(nf4, gf4, w1n4, w1s4, b1r, w24, sel, rep,
                        att_b2.reshape(1, 1))
    return jnp.concatenate(
        [lo.reshape(B, _PW), hi.reshape(B, _PW)], axis=1
    )
